# Initial kernel scaffold; baseline (speedup 1.0000x reference)
import jax
import jax.numpy as jnp
from jax.experimental import pallas as pl


def kernel(x, edge_index, batch, emb, W1l, b1l, W1r, W2l, b2l, W2r, Wlin, blin):
    raise NotImplementedError("write your pallas kernel here")


# speedup vs baseline: 7.4898x; 7.4898x over previous
import functools

import jax
import jax.numpy as jnp
from jax import lax
from jax.experimental import pallas as pl
from jax.experimental.pallas import tpu as pltpu
from jax.experimental.pallas import tpu_sc as plsc

N = 50000
E = 800000
V = 100000
D = 64
G = 1024

NC, NS = 2, 16
NW = NC * NS
NPAD = 50176
ROWS_W = NPAD // NW
EPAD = 802816
ER = EPAD // 128
DUMP = NPAD
SPROWS = NPAD + 16
ZSPAN = NPAD // NS
ZCH = 112
AROWS_W = ER // NW
BROWS_S = ER // NS
NBLK = NPAD // 256

_mesh = plsc.VectorSubcoreMesh(core_axis_name="c", subcore_axis_name="s",
                               num_cores=NC, num_subcores=NS)

_EMB_CHUNKS = [(j * 128, 128) for j in range(ROWS_W // 128)]
if ROWS_W % 128:
    _EMB_CHUNKS.append((ROWS_W - ROWS_W % 128, ROWS_W % 128))


@functools.partial(
    pl.kernel,
    out_type=(jax.ShapeDtypeStruct((NPAD, D), jnp.float32),
              jax.ShapeDtypeStruct((NC, NPAD, 16), jnp.float32)),
    mesh=_mesh,
    compiler_params=pltpu.CompilerParams(use_tc_tiling_on_sc=False),
    scratch_types=[
        pltpu.VMEM((ROWS_W,), jnp.int32),
        pltpu.VMEM((2, 128, D), jnp.float32),
        pltpu.VMEM((8, 128), jnp.int32),
        pltpu.VMEM((128, 16), jnp.float32),
        pltpu.VMEM((ZCH, 16), jnp.float32),
        pltpu.VMEM_SHARED((SPROWS, 16), jnp.float32),
        pltpu.SemaphoreType.DMA,
    ],
)
def _sc_embed_count(x_hbm, emb_hbm, dst_hbm, ones_hbm, z16_hbm,
                    h0_out, cnt_out, xv, ring, dbuf, onesv, zv, cnt_sp, sem):
    c = lax.axis_index("c")
    s = lax.axis_index("s")
    wid = c * NS + s
    base = wid * ROWS_W

    pltpu.sync_copy(z16_hbm, zv)
    for j in range(ZSPAN // ZCH):
        pltpu.sync_copy(zv, cnt_sp.at[pl.ds(s * ZSPAN + j * ZCH, ZCH), :])
    pltpu.sync_copy(ones_hbm, onesv)
    pltpu.sync_copy(x_hbm.at[pl.ds(base, ROWS_W)], xv)
    plsc.subcore_barrier()

    def fire(k):
        off, sz = _EMB_CHUNKS[k]
        d = pltpu.make_async_copy(emb_hbm.at[xv.at[pl.ds(off, sz)]],
                                  ring.at[k % 2, pl.ds(0, sz), :], sem)
        d.start()
        return d

    def drain(k, d):
        off, sz = _EMB_CHUNKS[k]
        d.wait()
        pltpu.sync_copy(ring.at[k % 2, pl.ds(0, sz), :],
                        h0_out.at[pl.ds(base + off, sz), :])

    prev = fire(0)
    for k in range(1, len(_EMB_CHUNKS)):
        cur = fire(k)
        drain(k - 1, prev)
        prev = cur
    drain(len(_EMB_CHUNKS) - 1, prev)

    ebase = wid * AROWS_W

    def count8(k0, _):
        pltpu.sync_copy(dst_hbm.at[pl.ds(ebase + k0 * 8, 8), :], dbuf)
        for j in range(8):
            pltpu.sync_copy(onesv, cnt_sp.at[dbuf.at[j]], add=True)
        return _
    lax.fori_loop(0, AROWS_W // 8, count8, None)
    rem = AROWS_W % 8
    if rem:
        pltpu.sync_copy(dst_hbm.at[pl.ds(ebase + AROWS_W - rem, rem), :],
                        dbuf.at[pl.ds(0, rem), :])
        for j in range(rem):
            pltpu.sync_copy(onesv, cnt_sp.at[dbuf.at[j]], add=True)
    plsc.subcore_barrier()

    pltpu.sync_copy(cnt_sp.at[pl.ds(s * ZSPAN, ZSPAN), :],
                    cnt_out.at[c, pl.ds(s * ZSPAN, ZSPAN), :])


@functools.partial(
    pl.kernel,
    out_type=jax.ShapeDtypeStruct((NC, NPAD, 32), jnp.float32),
    mesh=_mesh,
    compiler_params=pltpu.CompilerParams(use_tc_tiling_on_sc=False),
    scratch_types=[
        pltpu.VMEM((8, 128), jnp.int32),
        pltpu.VMEM((8, 128), jnp.int32),
        pltpu.VMEM((8, 128), jnp.int32),
        pltpu.VMEM((4, 128, 32), jnp.float32),
        pltpu.VMEM((ZCH, 32), jnp.float32),
        pltpu.VMEM_SHARED((SPROWS, 32), jnp.float32),
        pltpu.SemaphoreType.DMA,
    ],
)
def _sc_aggregate(h2v_hbm, src_hbm, dst_hbm, z32_hbm,
                  agg_out, sbuf, gidx, dbuf, rbuf, zv, agg_sp, sem):
    c = lax.axis_index("c")
    s = lax.axis_index("s")

    pltpu.sync_copy(z32_hbm, zv)
    for j in range(ZSPAN // ZCH):
        pltpu.sync_copy(zv, agg_sp.at[pl.ds(s * ZSPAN + j * ZCH, ZCH), :])
    plsc.subcore_barrier()

    rbase = s * BROWS_S

    def chunk(k0, _):
        row = rbase + k0 * 8
        pltpu.sync_copy(src_hbm.at[pl.ds(row, 8), :], sbuf)
        pltpu.sync_copy(dst_hbm.at[pl.ds(row, 8), :], dbuf)
        for j in range(8):
            for t in range(8):
                sv = sbuf[j, pl.ds(t * 16, 16)]
                gidx[j, pl.ds(t * 16, 16)] = sv * 2 + c

        def fire(j):
            pltpu.make_async_copy(h2v_hbm.at[gidx.at[j]],
                                  rbuf.at[j % 4], sem).start()

        def drain(j):
            pltpu.make_async_copy(h2v_hbm.at[gidx.at[j]],
                                  rbuf.at[j % 4], sem).wait()
            pltpu.sync_copy(rbuf.at[j % 4], agg_sp.at[dbuf.at[j]], add=True)

        for j in range(4):
            fire(j)
        for j in range(4, 8):
            drain(j - 4)
            fire(j)
        for j in range(4, 8):
            drain(j)
        return _

    lax.fori_loop(0, BROWS_S // 8, chunk, None)

    plsc.subcore_barrier()
    pltpu.sync_copy(agg_sp.at[pl.ds(s * ZSPAN, ZSPAN), :],
                    agg_out.at[c, pl.ds(s * ZSPAN, ZSPAN), :])


def _t1_body(h_ref, a0_ref, a1_ref, c0_ref, c1_ref, w_ref, b_ref, o_ref):
    cnt = c0_ref[0][:, 0:1] + c1_ref[0][:, 0:1]
    inv = 1.0 / jnp.maximum(cnt, 1.0)
    mean = jnp.concatenate([a0_ref[0], a1_ref[0]], axis=1) * inv
    hcat = jnp.concatenate([mean, h_ref[...]], axis=1)
    o_ref[...] = jnp.maximum(
        jnp.dot(hcat, w_ref[...], preferred_element_type=jnp.float32)
        + b_ref[...], 0.0)


def _t2_body(h_ref, a0_ref, a1_ref, c0_ref, c1_ref, bid_ref, w_ref, b_ref,
             wl_ref, bl_ref, o_ref, ps):
    i = pl.program_id(0)
    cnt = c0_ref[0][:, 0:1] + c1_ref[0][:, 0:1]
    inv = 1.0 / jnp.maximum(cnt, 1.0)
    mean = jnp.concatenate([a0_ref[0], a1_ref[0]], axis=1) * inv
    hcat = jnp.concatenate([mean, h_ref[...]], axis=1)
    h2 = jnp.maximum(
        jnp.dot(hcat, w_ref[...], preferred_element_type=jnp.float32)
        + b_ref[...], 0.0)

    bid = bid_ref[0, 0]
    onehot_t = (lax.broadcasted_iota(jnp.int32, (G, 256), 0)
                == bid[None, :]).astype(jnp.float32)
    hone = jnp.concatenate(
        [h2, jnp.ones((256, 1), jnp.float32), jnp.zeros((256, 63), jnp.float32)],
        axis=1)
    contrib = jnp.dot(onehot_t, hone, preferred_element_type=jnp.float32)

    @pl.when(i == 0)
    def _():
        ps[...] = contrib

    @pl.when(i > 0)
    def _():
        ps[...] = ps[...] + contrib

    @pl.when(i == NBLK - 1)
    def _():
        pooled = ps[:, 0:64] * (1.0 / jnp.maximum(ps[:, 64:65], 1.0))
        o_ref[...] = (jnp.dot(pooled, wl_ref[...],
                              preferred_element_type=jnp.float32) + bl_ref[...])


_b256 = pl.BlockSpec((256, D), lambda i: (i, 0))
_bagg0 = pl.BlockSpec((1, 256, 32), lambda i: (0, i, 0))
_bagg1 = pl.BlockSpec((1, 256, 32), lambda i: (1, i, 0))
_bcnt0 = pl.BlockSpec((1, 256, 16), lambda i: (0, i, 0))
_bcnt1 = pl.BlockSpec((1, 256, 16), lambda i: (1, i, 0))
_bw = pl.BlockSpec((2 * D, D), lambda i: (0, 0))
_bb = pl.BlockSpec((1, D), lambda i: (0, 0))

_t1 = pl.pallas_call(
    _t1_body,
    grid=(NBLK,),
    in_specs=[_b256, _bagg0, _bagg1, _bcnt0, _bcnt1, _bw, _bb],
    out_specs=_b256,
    out_shape=jax.ShapeDtypeStruct((NPAD, D), jnp.float32),
)

_t2 = pl.pallas_call(
    _t2_body,
    grid=(NBLK,),
    in_specs=[_b256, _bagg0, _bagg1, _bcnt0, _bcnt1,
              pl.BlockSpec((1, 1, 256), lambda i: (i, 0, 0)),
              _bw, _bb,
              pl.BlockSpec((D, 128), lambda i: (0, 0)),
              pl.BlockSpec((1, 128), lambda i: (0, 0))],
    out_specs=pl.BlockSpec((G, 128), lambda i: (0, 0)),
    out_shape=jax.ShapeDtypeStruct((G, 128), jnp.float32),
    scratch_shapes=[pltpu.VMEM((G, 128), jnp.float32)],
)


def kernel(x, edge_index, batch, emb, W1l, b1l, W1r, W2l, b2l, W2r, Wlin, blin):
    i32 = jnp.int32
    f32 = jnp.float32
    x_pad = jnp.concatenate([x, jnp.zeros((NPAD - N,), i32)])
    src_p = jnp.concatenate([edge_index[0], jnp.zeros((EPAD - E,), i32)])
    dst_p = jnp.concatenate([edge_index[1], jnp.full((EPAD - E,), DUMP, i32)])
    src_r = src_p.reshape(ER, 128)
    dst_r = dst_p.reshape(ER, 128)
    ones_c = jnp.ones((128, 16), f32)
    z16 = jnp.zeros((ZCH, 16), f32)
    z32 = jnp.zeros((ZCH, 32), f32)
    batch3 = jnp.concatenate([batch, jnp.full((NPAD - N,), G, i32)]
                             ).reshape(NBLK, 1, 256)
    w1cat = jnp.concatenate([W1l, W1r], axis=0)
    w2cat = jnp.concatenate([W2l, W2r], axis=0)
    wlin_pad = jnp.pad(Wlin, ((0, 0), (0, 128 - Wlin.shape[1])))
    blin_pad = jnp.pad(blin, (0, 128 - blin.shape[0])).reshape(1, 128)

    h0, cnt = _sc_embed_count(x_pad, emb, dst_r, ones_c, z16)
    agg1 = _sc_aggregate(h0.reshape(-1, 32), src_r, dst_r, z32)
    h1 = _t1(h0, agg1, agg1, cnt, cnt, w1cat, b1l.reshape(1, D))
    agg2 = _sc_aggregate(h1.reshape(-1, 32), src_r, dst_r, z32)
    out = _t2(h1, agg2, agg2, cnt, cnt, batch3, w2cat, b2l.reshape(1, D),
              wlin_pad, blin_pad)
    return out[:, :Wlin.shape[1]]


# speedup vs baseline: 8.9467x; 1.1945x over previous
import functools

import jax
import jax.numpy as jnp
from jax import lax
from jax.experimental import pallas as pl
from jax.experimental.pallas import tpu as pltpu
from jax.experimental.pallas import tpu_sc as plsc

N = 50000
E = 800000
V = 100000
D = 64
G = 1024

NC, NS = 2, 16
NW = NC * NS
NPAD = 50176
ROWS_W = NPAD // NW
EPAD = 802816
ER = EPAD // 128
DUMP = NPAD
SPROWS = NPAD + 16
ZSPAN = NPAD // NS
ZCH = 112
AROWS_W = ER // NW
BROWS_S = ER // NS
TBLK = 1024
NBLK = NPAD // TBLK

_mesh = plsc.VectorSubcoreMesh(core_axis_name="c", subcore_axis_name="s",
                               num_cores=NC, num_subcores=NS)

_EMB_CHUNKS = [(j * 128, 128) for j in range(ROWS_W // 128)]
if ROWS_W % 128:
    _EMB_CHUNKS.append((ROWS_W - ROWS_W % 128, ROWS_W % 128))


@functools.partial(
    pl.kernel,
    out_type=(jax.ShapeDtypeStruct((NPAD, D), jnp.float32),
              jax.ShapeDtypeStruct((NC, NPAD, 16), jnp.float32)),
    mesh=_mesh,
    compiler_params=pltpu.CompilerParams(use_tc_tiling_on_sc=False),
    scratch_types=[
        pltpu.VMEM((ROWS_W,), jnp.int32),
        pltpu.VMEM((2, 128, D), jnp.float32),
        pltpu.VMEM((8, 128), jnp.int32),
        pltpu.VMEM((128, 16), jnp.float32),
        pltpu.VMEM((ZCH, 16), jnp.float32),
        pltpu.VMEM_SHARED((SPROWS, 16), jnp.float32),
        pltpu.SemaphoreType.DMA,
    ],
)
def _sc_embed_count(x_hbm, emb_hbm, dst_hbm, ones_hbm, z16_hbm,
                    h0_out, cnt_out, xv, ring, dbuf, onesv, zv, cnt_sp, sem):
    c = lax.axis_index("c")
    s = lax.axis_index("s")
    wid = c * NS + s
    base = wid * ROWS_W

    pltpu.sync_copy(z16_hbm, zv)
    for j in range(ZSPAN // ZCH):
        pltpu.sync_copy(zv, cnt_sp.at[pl.ds(s * ZSPAN + j * ZCH, ZCH), :])
    pltpu.sync_copy(ones_hbm, onesv)
    pltpu.sync_copy(x_hbm.at[pl.ds(base, ROWS_W)], xv)
    plsc.subcore_barrier()

    def fire(k):
        off, sz = _EMB_CHUNKS[k]
        d = pltpu.make_async_copy(emb_hbm.at[xv.at[pl.ds(off, sz)]],
                                  ring.at[k % 2, pl.ds(0, sz), :], sem)
        d.start()
        return d

    def drain(k, d):
        off, sz = _EMB_CHUNKS[k]
        d.wait()
        pltpu.sync_copy(ring.at[k % 2, pl.ds(0, sz), :],
                        h0_out.at[pl.ds(base + off, sz), :])

    prev = fire(0)
    for k in range(1, len(_EMB_CHUNKS)):
        cur = fire(k)
        drain(k - 1, prev)
        prev = cur
    drain(len(_EMB_CHUNKS) - 1, prev)

    ebase = wid * AROWS_W

    def count8(k0, _):
        pltpu.sync_copy(dst_hbm.at[pl.ds(ebase + k0 * 8, 8), :], dbuf)
        for j in range(8):
            pltpu.sync_copy(onesv, cnt_sp.at[dbuf.at[j]], add=True)
        return _
    lax.fori_loop(0, AROWS_W // 8, count8, None)
    rem = AROWS_W % 8
    if rem:
        pltpu.sync_copy(dst_hbm.at[pl.ds(ebase + AROWS_W - rem, rem), :],
                        dbuf.at[pl.ds(0, rem), :])
        for j in range(rem):
            pltpu.sync_copy(onesv, cnt_sp.at[dbuf.at[j]], add=True)
    plsc.subcore_barrier()

    pltpu.sync_copy(cnt_sp.at[pl.ds(s * ZSPAN, ZSPAN), :],
                    cnt_out.at[c, pl.ds(s * ZSPAN, ZSPAN), :])


@functools.partial(
    pl.kernel,
    out_type=jax.ShapeDtypeStruct((NC, NPAD, 32), jnp.float32),
    mesh=_mesh,
    compiler_params=pltpu.CompilerParams(use_tc_tiling_on_sc=False),
    scratch_types=[
        pltpu.VMEM((8, 128), jnp.int32),
        pltpu.VMEM((8, 128), jnp.int32),
        pltpu.VMEM((8, 128), jnp.int32),
        pltpu.VMEM((4, 128, 32), jnp.float32),
        pltpu.VMEM((ZCH, 32), jnp.float32),
        pltpu.VMEM_SHARED((SPROWS, 32), jnp.float32),
        pltpu.SemaphoreType.DMA,
    ],
)
def _sc_aggregate(h2v_hbm, src_hbm, dst_hbm, z32_hbm,
                  agg_out, sbuf, gidx, dbuf, rbuf, zv, agg_sp, sem):
    c = lax.axis_index("c")
    s = lax.axis_index("s")

    pltpu.sync_copy(z32_hbm, zv)
    for j in range(ZSPAN // ZCH):
        pltpu.sync_copy(zv, agg_sp.at[pl.ds(s * ZSPAN + j * ZCH, ZCH), :])
    plsc.subcore_barrier()

    rbase = s * BROWS_S

    def chunk(k0, _):
        row = rbase + k0 * 8
        pltpu.sync_copy(src_hbm.at[pl.ds(row, 8), :], sbuf)
        pltpu.sync_copy(dst_hbm.at[pl.ds(row, 8), :], dbuf)
        for j in range(8):
            for t in range(8):
                sv = sbuf[j, pl.ds(t * 16, 16)]
                gidx[j, pl.ds(t * 16, 16)] = sv * 2 + c

        def fire(j):
            pltpu.make_async_copy(h2v_hbm.at[gidx.at[j]],
                                  rbuf.at[j % 4], sem).start()

        def drain(j):
            pltpu.make_async_copy(h2v_hbm.at[gidx.at[j]],
                                  rbuf.at[j % 4], sem).wait()
            pltpu.sync_copy(rbuf.at[j % 4], agg_sp.at[dbuf.at[j]], add=True)

        for j in range(4):
            fire(j)
        for j in range(4, 8):
            drain(j - 4)
            fire(j)
        for j in range(4, 8):
            drain(j)
        return _

    lax.fori_loop(0, BROWS_S // 8, chunk, None)

    plsc.subcore_barrier()
    pltpu.sync_copy(agg_sp.at[pl.ds(s * ZSPAN, ZSPAN), :],
                    agg_out.at[c, pl.ds(s * ZSPAN, ZSPAN), :])


def _t1_body(h_ref, a0_ref, a1_ref, c0_ref, c1_ref, w_ref, b_ref, o_ref):
    cnt = c0_ref[0][:, 0:1] + c1_ref[0][:, 0:1]
    inv = 1.0 / jnp.maximum(cnt, 1.0)
    mean = jnp.concatenate([a0_ref[0], a1_ref[0]], axis=1) * inv
    hcat = jnp.concatenate([mean, h_ref[...]], axis=1)
    o_ref[...] = jnp.maximum(
        jnp.dot(hcat, w_ref[...], preferred_element_type=jnp.float32)
        + b_ref[...], 0.0)


def _t2_body(h_ref, a0_ref, a1_ref, c0_ref, c1_ref, bid_ref, w_ref, b_ref,
             wl_ref, bl_ref, o_ref, ps):
    i = pl.program_id(0)
    cnt = c0_ref[0][:, 0:1] + c1_ref[0][:, 0:1]
    inv = 1.0 / jnp.maximum(cnt, 1.0)
    mean = jnp.concatenate([a0_ref[0], a1_ref[0]], axis=1) * inv
    hcat = jnp.concatenate([mean, h_ref[...]], axis=1)
    h2 = jnp.maximum(
        jnp.dot(hcat, w_ref[...], preferred_element_type=jnp.float32)
        + b_ref[...], 0.0)

    bid = bid_ref[0, 0]
    onehot_t = (lax.broadcasted_iota(jnp.int32, (G, TBLK), 0)
                == bid[None, :]).astype(jnp.float32)
    hone = jnp.concatenate(
        [h2, jnp.ones((TBLK, 1), jnp.float32),
         jnp.zeros((TBLK, 63), jnp.float32)],
        axis=1)
    contrib = jnp.dot(onehot_t, hone, preferred_element_type=jnp.float32)

    @pl.when(i == 0)
    def _():
        ps[...] = contrib

    @pl.when(i > 0)
    def _():
        ps[...] = ps[...] + contrib

    @pl.when(i == NBLK - 1)
    def _():
        pooled = ps[:, 0:64] * (1.0 / jnp.maximum(ps[:, 64:65], 1.0))
        o_ref[...] = (jnp.dot(pooled, wl_ref[...],
                              preferred_element_type=jnp.float32) + bl_ref[...])


_b256 = pl.BlockSpec((TBLK, D), lambda i: (i, 0))
_bagg0 = pl.BlockSpec((1, TBLK, 32), lambda i: (0, i, 0))
_bagg1 = pl.BlockSpec((1, TBLK, 32), lambda i: (1, i, 0))
_bcnt0 = pl.BlockSpec((1, TBLK, 16), lambda i: (0, i, 0))
_bcnt1 = pl.BlockSpec((1, TBLK, 16), lambda i: (1, i, 0))
_bw = pl.BlockSpec((2 * D, D), lambda i: (0, 0))
_bb = pl.BlockSpec((1, D), lambda i: (0, 0))

_t1 = pl.pallas_call(
    _t1_body,
    grid=(NBLK,),
    in_specs=[_b256, _bagg0, _bagg1, _bcnt0, _bcnt1, _bw, _bb],
    out_specs=_b256,
    out_shape=jax.ShapeDtypeStruct((NPAD, D), jnp.float32),
)

_t2 = pl.pallas_call(
    _t2_body,
    grid=(NBLK,),
    in_specs=[_b256, _bagg0, _bagg1, _bcnt0, _bcnt1,
              pl.BlockSpec((1, 1, TBLK), lambda i: (i, 0, 0)),
              _bw, _bb,
              pl.BlockSpec((D, 128), lambda i: (0, 0)),
              pl.BlockSpec((1, 128), lambda i: (0, 0))],
    out_specs=pl.BlockSpec((G, 128), lambda i: (0, 0)),
    out_shape=jax.ShapeDtypeStruct((G, 128), jnp.float32),
    scratch_shapes=[pltpu.VMEM((G, 128), jnp.float32)],
)


def kernel(x, edge_index, batch, emb, W1l, b1l, W1r, W2l, b2l, W2r, Wlin, blin):
    i32 = jnp.int32
    f32 = jnp.float32
    x_pad = jnp.concatenate([x, jnp.zeros((NPAD - N,), i32)])
    src_p = jnp.concatenate([edge_index[0], jnp.zeros((EPAD - E,), i32)])
    dst_p = jnp.concatenate([edge_index[1], jnp.full((EPAD - E,), DUMP, i32)])
    src_r = src_p.reshape(ER, 128)
    dst_r = dst_p.reshape(ER, 128)
    ones_c = jnp.ones((128, 16), f32)
    z16 = jnp.zeros((ZCH, 16), f32)
    z32 = jnp.zeros((ZCH, 32), f32)
    batch3 = jnp.concatenate([batch, jnp.full((NPAD - N,), G, i32)]
                             ).reshape(NBLK, 1, TBLK)
    w1cat = jnp.concatenate([W1l, W1r], axis=0)
    w2cat = jnp.concatenate([W2l, W2r], axis=0)
    wlin_pad = jnp.pad(Wlin, ((0, 0), (0, 128 - Wlin.shape[1])))
    blin_pad = jnp.pad(blin, (0, 128 - blin.shape[0])).reshape(1, 128)

    h0, cnt = _sc_embed_count(x_pad, emb, dst_r, ones_c, z16)
    agg1 = _sc_aggregate(h0.reshape(-1, 32), src_r, dst_r, z32)
    h1 = _t1(h0, agg1, agg1, cnt, cnt, w1cat, b1l.reshape(1, D))
    agg2 = _sc_aggregate(h1.reshape(-1, 32), src_r, dst_r, z32)
    out = _t2(h1, agg2, agg2, cnt, cnt, batch3, w2cat, b2l.reshape(1, D),
              wlin_pad, blin_pad)
    return out[:, :Wlin.shape[1]]


# speedup vs baseline: 10.0117x; 1.1190x over previous
import functools

import jax
import jax.numpy as jnp
from jax import lax
from jax.experimental import pallas as pl
from jax.experimental.pallas import tpu as pltpu
from jax.experimental.pallas import tpu_sc as plsc

N = 50000
E = 800000
V = 100000
D = 64
G = 1024

NC, NS = 2, 16
NW = NC * NS
NPAD = 50176
ROWS_W = NPAD // NW
EROWS = E // 128
SPROWS = NPAD + 16
ZSPAN = NPAD // NS
ZCH = 112
A_STRIDE = 200
B_STRIDE = 392
TBLK = 1024
NBLK = NPAD // TBLK

_mesh = plsc.VectorSubcoreMesh(core_axis_name="c", subcore_axis_name="s",
                               num_cores=NC, num_subcores=NS)

_EMB_CHUNKS = [(j * 128, 128) for j in range(ROWS_W // 128)]
if ROWS_W % 128:
    _EMB_CHUNKS.append((ROWS_W - ROWS_W % 128, ROWS_W % 128))


@functools.partial(
    pl.kernel,
    out_type=(jax.ShapeDtypeStruct((NPAD, D), jnp.float32),
              jax.ShapeDtypeStruct((NC, NPAD, 16), jnp.float32)),
    mesh=_mesh,
    compiler_params=pltpu.CompilerParams(use_tc_tiling_on_sc=False),
    scratch_types=[
        pltpu.VMEM((ROWS_W,), jnp.int32),
        pltpu.VMEM((2, 128, D), jnp.float32),
        pltpu.VMEM((1024,), jnp.int32),
        pltpu.VMEM((128, 16), jnp.float32),
        pltpu.VMEM((ZCH, 16), jnp.float32),
        pltpu.VMEM_SHARED((SPROWS, 16), jnp.float32),
        pltpu.SemaphoreType.DMA,
    ],
)
def _sc_embed_count(x_hbm, emb_hbm, edge_hbm, ones_hbm, z16_hbm,
                    h0_out, cnt_out, xv, ring, dbuf, onesv, zv, cnt_sp, sem):
    c = lax.axis_index("c")
    s = lax.axis_index("s")
    wid = c * NS + s
    base = wid * ROWS_W

    pltpu.sync_copy(z16_hbm, zv)
    for j in range(ZSPAN // ZCH):
        pltpu.sync_copy(zv, cnt_sp.at[pl.ds(s * ZSPAN + j * ZCH, ZCH), :])
    pltpu.sync_copy(ones_hbm, onesv)
    pltpu.sync_copy(x_hbm.at[pl.ds(base, ROWS_W)], xv)
    plsc.subcore_barrier()

    def fire(k):
        off, sz = _EMB_CHUNKS[k]
        d = pltpu.make_async_copy(emb_hbm.at[xv.at[pl.ds(off, sz)]],
                                  ring.at[k % 2, pl.ds(0, sz), :], sem)
        d.start()
        return d

    def drain(k, d):
        off, sz = _EMB_CHUNKS[k]
        d.wait()
        pltpu.sync_copy(ring.at[k % 2, pl.ds(0, sz), :],
                        h0_out.at[pl.ds(base + off, sz), :])

    prev = fire(0)
    for k in range(1, len(_EMB_CHUNKS)):
        cur = fire(k)
        drain(k - 1, prev)
        prev = cur
    drain(len(_EMB_CHUNKS) - 1, prev)

    ebase = wid * A_STRIDE
    ecnt = jnp.minimum(jnp.maximum(EROWS - ebase, 0), A_STRIDE)
    nfull = ecnt // 8
    ntail = ecnt - nfull * 8

    def count8(k0, _):
        off = (ebase + k0 * 8) * 128
        pltpu.sync_copy(edge_hbm.at[1, pl.ds(off, 1024)], dbuf)
        for j in range(8):
            pltpu.sync_copy(onesv, cnt_sp.at[dbuf.at[pl.ds(j * 128, 128)]],
                            add=True)
        return _
    lax.fori_loop(0, nfull, count8, None)

    def count1(j, _):
        off = (ebase + nfull * 8 + j) * 128
        pltpu.sync_copy(edge_hbm.at[1, pl.ds(off, 128)],
                        dbuf.at[pl.ds(0, 128)])
        pltpu.sync_copy(onesv, cnt_sp.at[dbuf.at[pl.ds(0, 128)]], add=True)
        return _
    lax.fori_loop(0, ntail, count1, None)
    plsc.subcore_barrier()

    pltpu.sync_copy(cnt_sp.at[pl.ds(s * ZSPAN, ZSPAN), :],
                    cnt_out.at[c, pl.ds(s * ZSPAN, ZSPAN), :])


@functools.partial(
    pl.kernel,
    out_type=jax.ShapeDtypeStruct((NC, NPAD, 32), jnp.float32),
    mesh=_mesh,
    compiler_params=pltpu.CompilerParams(use_tc_tiling_on_sc=False),
    scratch_types=[
        pltpu.VMEM((1024,), jnp.int32),
        pltpu.VMEM((1024,), jnp.int32),
        pltpu.VMEM((1024,), jnp.int32),
        pltpu.VMEM((4, 128, 32), jnp.float32),
        pltpu.VMEM((ZCH, 32), jnp.float32),
        pltpu.VMEM_SHARED((SPROWS, 32), jnp.float32),
        pltpu.SemaphoreType.DMA,
    ],
)
def _sc_aggregate(h2v_hbm, edge_hbm, z32_hbm,
                  agg_out, sbuf, gidx, dbuf, rbuf, zv, agg_sp, sem):
    c = lax.axis_index("c")
    s = lax.axis_index("s")

    pltpu.sync_copy(z32_hbm, zv)
    for j in range(ZSPAN // ZCH):
        pltpu.sync_copy(zv, agg_sp.at[pl.ds(s * ZSPAN + j * ZCH, ZCH), :])
    plsc.subcore_barrier()

    ebase = s * B_STRIDE
    ecnt = jnp.minimum(jnp.maximum(EROWS - ebase, 0), B_STRIDE)
    nfull = ecnt // 8
    ntail = ecnt - nfull * 8

    def chunk(k0, _):
        off = (ebase + k0 * 8) * 128
        pltpu.sync_copy(edge_hbm.at[0, pl.ds(off, 1024)], sbuf)
        pltpu.sync_copy(edge_hbm.at[1, pl.ds(off, 1024)], dbuf)
        for t in range(64):
            sv = sbuf[pl.ds(t * 16, 16)]
            gidx[pl.ds(t * 16, 16)] = sv * 2 + c

        def fire(j):
            pltpu.make_async_copy(h2v_hbm.at[gidx.at[pl.ds(j * 128, 128)]],
                                  rbuf.at[j % 4], sem).start()

        def drain(j):
            pltpu.make_async_copy(h2v_hbm.at[gidx.at[pl.ds(j * 128, 128)]],
                                  rbuf.at[j % 4], sem).wait()
            pltpu.sync_copy(rbuf.at[j % 4],
                            agg_sp.at[dbuf.at[pl.ds(j * 128, 128)]], add=True)

        for j in range(4):
            fire(j)
        for j in range(4, 8):
            drain(j - 4)
            fire(j)
        for j in range(4, 8):
            drain(j)
        return _

    lax.fori_loop(0, nfull, chunk, None)

    def tailrow(j, _):
        off = (ebase + nfull * 8 + j) * 128
        pltpu.sync_copy(edge_hbm.at[0, pl.ds(off, 128)],
                        sbuf.at[pl.ds(0, 128)])
        pltpu.sync_copy(edge_hbm.at[1, pl.ds(off, 128)],
                        dbuf.at[pl.ds(0, 128)])
        for t in range(8):
            sv = sbuf[pl.ds(t * 16, 16)]
            gidx[pl.ds(t * 16, 16)] = sv * 2 + c
        d = pltpu.make_async_copy(h2v_hbm.at[gidx.at[pl.ds(0, 128)]],
                                  rbuf.at[0], sem)
        d.start()
        d.wait()
        pltpu.sync_copy(rbuf.at[0], agg_sp.at[dbuf.at[pl.ds(0, 128)]],
                        add=True)
        return _
    lax.fori_loop(0, ntail, tailrow, None)

    plsc.subcore_barrier()
    pltpu.sync_copy(agg_sp.at[pl.ds(s * ZSPAN, ZSPAN), :],
                    agg_out.at[c, pl.ds(s * ZSPAN, ZSPAN), :])


def _t1_body(h_ref, a0_ref, a1_ref, c0_ref, c1_ref, w_ref, b_ref, o_ref):
    cnt = c0_ref[0][:, 0:1] + c1_ref[0][:, 0:1]
    inv = 1.0 / jnp.maximum(cnt, 1.0)
    mean = jnp.concatenate([a0_ref[0], a1_ref[0]], axis=1) * inv
    hcat = jnp.concatenate([mean, h_ref[...]], axis=1)
    o_ref[...] = jnp.maximum(
        jnp.dot(hcat, w_ref[...], preferred_element_type=jnp.float32)
        + b_ref[...], 0.0)


def _t2_body(h_ref, a0_ref, a1_ref, c0_ref, c1_ref, bid_ref, w_ref, b_ref,
             wl_ref, bl_ref, o_ref, ps):
    i = pl.program_id(0)
    cnt = c0_ref[0][:, 0:1] + c1_ref[0][:, 0:1]
    inv = 1.0 / jnp.maximum(cnt, 1.0)
    mean = jnp.concatenate([a0_ref[0], a1_ref[0]], axis=1) * inv
    hcat = jnp.concatenate([mean, h_ref[...]], axis=1)
    h2 = jnp.maximum(
        jnp.dot(hcat, w_ref[...], preferred_element_type=jnp.float32)
        + b_ref[...], 0.0)

    bid = bid_ref[0, 0]
    onehot_t = (lax.broadcasted_iota(jnp.int32, (G, TBLK), 0)
                == bid[None, :]).astype(jnp.float32)
    hone = jnp.concatenate(
        [h2, jnp.ones((TBLK, 1), jnp.float32),
         jnp.zeros((TBLK, 63), jnp.float32)],
        axis=1)
    contrib = jnp.dot(onehot_t, hone, preferred_element_type=jnp.float32)

    @pl.when(i == 0)
    def _():
        ps[...] = contrib

    @pl.when(i > 0)
    def _():
        ps[...] = ps[...] + contrib

    @pl.when(i == NBLK - 1)
    def _():
        pooled = ps[:, 0:64] * (1.0 / jnp.maximum(ps[:, 64:65], 1.0))
        o_ref[...] = (jnp.dot(pooled, wl_ref[...],
                              preferred_element_type=jnp.float32) + bl_ref[...])


_b256 = pl.BlockSpec((TBLK, D), lambda i: (i, 0))
_bagg0 = pl.BlockSpec((1, TBLK, 32), lambda i: (0, i, 0))
_bagg1 = pl.BlockSpec((1, TBLK, 32), lambda i: (1, i, 0))
_bcnt0 = pl.BlockSpec((1, TBLK, 16), lambda i: (0, i, 0))
_bcnt1 = pl.BlockSpec((1, TBLK, 16), lambda i: (1, i, 0))
_bw = pl.BlockSpec((2 * D, D), lambda i: (0, 0))
_bb = pl.BlockSpec((1, D), lambda i: (0, 0))

_t1 = pl.pallas_call(
    _t1_body,
    grid=(NBLK,),
    in_specs=[_b256, _bagg0, _bagg1, _bcnt0, _bcnt1, _bw, _bb],
    out_specs=_b256,
    out_shape=jax.ShapeDtypeStruct((NPAD, D), jnp.float32),
)

_t2 = pl.pallas_call(
    _t2_body,
    grid=(NBLK,),
    in_specs=[_b256, _bagg0, _bagg1, _bcnt0, _bcnt1,
              pl.BlockSpec((1, 1, TBLK), lambda i: (i, 0, 0)),
              _bw, _bb,
              pl.BlockSpec((D, 128), lambda i: (0, 0)),
              pl.BlockSpec((1, 128), lambda i: (0, 0))],
    out_specs=pl.BlockSpec((G, 128), lambda i: (0, 0)),
    out_shape=jax.ShapeDtypeStruct((G, 128), jnp.float32),
    scratch_shapes=[pltpu.VMEM((G, 128), jnp.float32)],
)


def kernel(x, edge_index, batch, emb, W1l, b1l, W1r, W2l, b2l, W2r, Wlin, blin):
    i32 = jnp.int32
    f32 = jnp.float32
    x_pad = jnp.concatenate([x, jnp.zeros((NPAD - N,), i32)])
    ones_c = jnp.ones((128, 16), f32)
    z16 = jnp.zeros((ZCH, 16), f32)
    z32 = jnp.zeros((ZCH, 32), f32)
    batch3 = jnp.concatenate([batch, jnp.full((NPAD - N,), G, i32)]
                             ).reshape(NBLK, 1, TBLK)
    w1cat = jnp.concatenate([W1l, W1r], axis=0)
    w2cat = jnp.concatenate([W2l, W2r], axis=0)
    wlin_pad = jnp.pad(Wlin, ((0, 0), (0, 128 - Wlin.shape[1])))
    blin_pad = jnp.pad(blin, (0, 128 - blin.shape[0])).reshape(1, 128)

    h0, cnt = _sc_embed_count(x_pad, emb, edge_index, ones_c, z16)
    agg1 = _sc_aggregate(h0.reshape(-1, 32), edge_index, z32)
    h1 = _t1(h0, agg1, agg1, cnt, cnt, w1cat, b1l.reshape(1, D))
    agg2 = _sc_aggregate(h1.reshape(-1, 32), edge_index, z32)
    out = _t2(h1, agg2, agg2, cnt, cnt, batch3, w2cat, b2l.reshape(1, D),
              wlin_pad, blin_pad)
    return out[:, :Wlin.shape[1]]
